# baseline (speedup 1.0000x reference)
import functools
import math

import numpy as np
import jax
import jax.numpy as jnp
from jax import lax
from jax.experimental import pallas as pl
from jax.experimental.pallas import tpu as pltpu
from jax.experimental.pallas import tpu_sc as plsc

_MAX_LEN = 2048
_D = 2048
_B = 2
_L = 2048

_NC = 2
_NS = 16
_NW = _NC * _NS
_LPW = _L // _NW
_CH = 8
_NCHUNK = _LPW // _CH
_LANES = 16


def _pos_table_np() -> np.ndarray:
    pos = np.arange(_MAX_LEN, dtype=np.float32)[:, None]
    div = np.exp(
        np.arange(0, _D, 2, dtype=np.float32) * np.float32(-math.log(10000.0) / _D)
    )
    ang = pos * div
    emb = np.zeros((_MAX_LEN, _D), dtype=np.float32)
    emb[:, 0::2] = np.sin(ang)
    emb[:, 1::2] = np.cos(ang)
    return emb


_POS = _pos_table_np()

_mesh = plsc.VectorSubcoreMesh(core_axis_name="c", subcore_axis_name="s")


@functools.partial(
    pl.kernel,
    mesh=_mesh,
    out_type=jax.ShapeDtypeStruct((_B * _L, _D), jnp.float32),
    scratch_types=[
        pltpu.VMEM((_B, _LPW), jnp.int32),
        pltpu.VMEM((_CH, _D), jnp.float32),
        pltpu.VMEM((_CH, _D), jnp.float32),
        pltpu.VMEM((_CH, _D), jnp.float32),
        pltpu.VMEM((_CH, _D), jnp.float32),
        pltpu.VMEM((_CH, _D), jnp.float32),
        pltpu.VMEM((_CH, _D), jnp.float32),
        pltpu.SemaphoreType.DMA,
        pltpu.SemaphoreType.DMA,
        pltpu.SemaphoreType.DMA,
        pltpu.SemaphoreType.DMA,
        pltpu.SemaphoreType.DMA,
        pltpu.SemaphoreType.DMA,
        pltpu.SemaphoreType.DMA,
        pltpu.SemaphoreType.DMA,
    ],
)
def _emb_lookup(xf_hbm, table_hbm, pos_hbm, out_hbm, idx_v,
                rows0_a, rows1_a, pos_a, rows0_b, rows1_b, pos_b,
                sem_g0_a, sem_g1_a, sem_p_a, sem_g0_b, sem_g1_b, sem_p_b,
                sem_out_a, sem_out_b):
    wid = lax.axis_index("s") * _NC + lax.axis_index("c")
    base = wid * _LPW

    pltpu.sync_copy(xf_hbm.at[pl.ds(base, _LPW)], idx_v.at[0])
    pltpu.sync_copy(xf_hbm.at[pl.ds(_L + base, _LPW)], idx_v.at[1])

    sets = (
        (rows0_a, rows1_a, pos_a, sem_g0_a, sem_g1_a, sem_p_a, sem_out_a),
        (rows0_b, rows1_b, pos_b, sem_g0_b, sem_g1_b, sem_p_b, sem_out_b),
    )

    def start_in(c):
        rows0, rows1, posb, sem_g0, sem_g1, sem_p, _ = sets[c % 2]
        g0 = pltpu.async_copy(
            table_hbm.at[idx_v.at[0, pl.ds(c * _CH, _CH)]], rows0, sem_g0)
        g1 = pltpu.async_copy(
            table_hbm.at[idx_v.at[1, pl.ds(c * _CH, _CH)]], rows1, sem_g1)
        p = pltpu.async_copy(
            pos_hbm.at[pl.ds(base + c * _CH, _CH)], posb, sem_p)
        return (g0, g1, p)

    def start_out(c, which, rows):
        _, _, _, _, _, _, sem_out = sets[c % 2]
        return pltpu.async_copy(
            rows, out_hbm.at[pl.ds(which * _L + base + c * _CH, _CH)], sem_out)

    def add_pos(c):
        rows0, rows1, posb = sets[c % 2][:3]

        def add_body(j, carry):
            for dj in range(2):
                col = (j * 2 + dj) * _LANES
                for r in range(_CH):
                    pv = posb[r, pl.ds(col, _LANES)]
                    plsc.addupdate(rows0.at[r, pl.ds(col, _LANES)], pv)
                    plsc.addupdate(rows1.at[r, pl.ds(col, _LANES)], pv)
            return carry

        lax.fori_loop(0, _D // (2 * _LANES), add_body, 0)

    in_descs = {0: start_in(0)}
    out_descs = {}
    for c in range(_NCHUNK):
        rows0, rows1, posb = sets[c % 2][:3]
        if c + 1 < _NCHUNK:
            if c >= 1:
                for d in out_descs.pop(c - 1):
                    d.wait()
            in_descs[c + 1] = start_in(c + 1)
        g0, g1, p = in_descs.pop(c)
        p.wait()
        g0.wait()
        g1.wait()
        add_pos(c)
        out_descs[c] = (start_out(c, 0, rows0), start_out(c, 1, rows1))
    for c in (_NCHUNK - 2, _NCHUNK - 1):
        for d in out_descs.pop(c):
            d.wait()


def kernel(x, token_table):
    xf = x.reshape(-1).astype(jnp.int32)
    pos = jnp.asarray(_POS)
    out = _emb_lookup(xf, token_table, pos)
    return out.reshape(_B, _L, _D)


# speedup vs baseline: 1.2373x; 1.2373x over previous
import functools
import math

import numpy as np
import jax
import jax.numpy as jnp
from jax import lax
from jax.experimental import pallas as pl
from jax.experimental.pallas import tpu as pltpu
from jax.experimental.pallas import tpu_sc as plsc

_MAX_LEN = 2048
_D = 2048
_B = 2
_L = 2048

_NC = 2
_NS = 16
_NW = _NC * _NS
_LPW = _L // _NW
_CH = 4
_NCHUNK = _LPW // _CH
_NSET = 4
_LANES = 16


def _pos_table_np() -> np.ndarray:
    pos = np.arange(_MAX_LEN, dtype=np.float32)[:, None]
    div = np.exp(
        np.arange(0, _D, 2, dtype=np.float32) * np.float32(-math.log(10000.0) / _D)
    )
    ang = pos * div
    emb = np.zeros((_MAX_LEN, _D), dtype=np.float32)
    emb[:, 0::2] = np.sin(ang)
    emb[:, 1::2] = np.cos(ang)
    return emb


_POS = _pos_table_np()

_mesh = plsc.VectorSubcoreMesh(core_axis_name="c", subcore_axis_name="s")


@functools.partial(
    pl.kernel,
    mesh=_mesh,
    out_type=jax.ShapeDtypeStruct((_B * _L, _D), jnp.float32),
    scratch_types=(
        [pltpu.VMEM((_B, _LPW), jnp.int32)]
        + [pltpu.VMEM((_CH, _D), jnp.float32) for _ in range(3 * _NSET)]
        + [pltpu.SemaphoreType.DMA for _ in range(2 * _NSET)]
    ),
)
def _emb_lookup(xf_hbm, table_hbm, pos_hbm, out_hbm, idx_v, *bufs):
    wid = lax.axis_index("s") * _NC + lax.axis_index("c")
    base = wid * _LPW

    pltpu.sync_copy(xf_hbm.at[pl.ds(base, _LPW)], idx_v.at[0])
    pltpu.sync_copy(xf_hbm.at[pl.ds(_L + base, _LPW)], idx_v.at[1])

    vmem = bufs[: 3 * _NSET]
    sems = bufs[3 * _NSET:]
    sets = tuple(
        (vmem[3 * s], vmem[3 * s + 1], vmem[3 * s + 2], sems[2 * s], sems[2 * s + 1])
        for s in range(_NSET)
    )

    def in_descs(c, s):
        rows0, rows1, posb, sem_in, _ = sets[s]
        g0 = pltpu.make_async_copy(
            table_hbm.at[idx_v.at[0, pl.ds(c * _CH, _CH)]], rows0, sem_in)
        g1 = pltpu.make_async_copy(
            table_hbm.at[idx_v.at[1, pl.ds(c * _CH, _CH)]], rows1, sem_in)
        p = pltpu.make_async_copy(
            pos_hbm.at[pl.ds(base + c * _CH, _CH)], posb, sem_in)
        return (g0, g1, p)

    def out_descs(c, s):
        rows0, rows1, _, _, sem_out = sets[s]
        o0 = pltpu.make_async_copy(
            rows0, out_hbm.at[pl.ds(base + c * _CH, _CH)], sem_out)
        o1 = pltpu.make_async_copy(
            rows1, out_hbm.at[pl.ds(_L + base + c * _CH, _CH)], sem_out)
        return (o0, o1)

    def add_pos(s):
        rows0, rows1, posb, _, _ = sets[s]

        def add_body(j, carry):
            col = j * _LANES
            for r in range(_CH):
                pv = posb[r, pl.ds(col, _LANES)]
                plsc.addupdate(rows0.at[r, pl.ds(col, _LANES)], pv)
                plsc.addupdate(rows1.at[r, pl.ds(col, _LANES)], pv)
            return carry

        lax.fori_loop(0, _D // _LANES, add_body, 0)

    for d in in_descs(0, 0):
        d.start()
    for d in in_descs(1, 1):
        d.start()

    def quad_body(q, carry):
        for s in range(_NSET):
            c = q * _NSET + s
            s2 = (s + 2) % _NSET

            @pl.when(c >= 2)
            def _():
                for d in out_descs(c - 2, s2):
                    d.wait()

            @pl.when(c + 2 < _NCHUNK)
            def _():
                for d in in_descs(c + 2, s2):
                    d.start()

            for d in in_descs(c, s):
                d.wait()
            add_pos(s)
            for d in out_descs(c, s):
                d.start()
        return carry

    lax.fori_loop(0, _NCHUNK // _NSET, quad_body, 0)

    for d in out_descs(_NCHUNK - 2, (_NCHUNK - 2) % _NSET):
        d.wait()
    for d in out_descs(_NCHUNK - 1, (_NCHUNK - 1) % _NSET):
        d.wait()


def kernel(x, token_table):
    xf = x.reshape(-1).astype(jnp.int32)
    pos = jnp.asarray(_POS)
    out = _emb_lookup(xf, token_table, pos)
    return out.reshape(_B, _L, _D)


# speedup vs baseline: 1.2760x; 1.0312x over previous
import functools
import math

import numpy as np
import jax
import jax.numpy as jnp
from jax import lax
from jax.experimental import pallas as pl
from jax.experimental.pallas import tpu as pltpu
from jax.experimental.pallas import tpu_sc as plsc

_MAX_LEN = 2048
_D = 2048
_B = 2
_L = 2048

_NC = 2
_NS = 16
_NW = _NC * _NS
_LPW = _L // _NW
_CH = 2
_NCHUNK = _LPW // _CH
_NSET = 8
_LEAD = _NSET // 2
_LANES = 16


def _pos_table_np() -> np.ndarray:
    pos = np.arange(_MAX_LEN, dtype=np.float32)[:, None]
    div = np.exp(
        np.arange(0, _D, 2, dtype=np.float32) * np.float32(-math.log(10000.0) / _D)
    )
    ang = pos * div
    emb = np.zeros((_MAX_LEN, _D), dtype=np.float32)
    emb[:, 0::2] = np.sin(ang)
    emb[:, 1::2] = np.cos(ang)
    return emb


_POS = _pos_table_np()

_mesh = plsc.VectorSubcoreMesh(core_axis_name="c", subcore_axis_name="s")


@functools.partial(
    pl.kernel,
    mesh=_mesh,
    out_type=jax.ShapeDtypeStruct((_B * _L, _D), jnp.float32),
    scratch_types=(
        [pltpu.VMEM((_B, _LPW), jnp.int32)]
        + [pltpu.VMEM((_CH, _D), jnp.float32) for _ in range(3 * _NSET)]
        + [pltpu.SemaphoreType.DMA for _ in range(2 * _NSET)]
    ),
)
def _emb_lookup(xf_hbm, table_hbm, pos_hbm, out_hbm, idx_v, *bufs):
    wid = lax.axis_index("s") * _NC + lax.axis_index("c")
    base = wid * _LPW

    pltpu.sync_copy(xf_hbm.at[pl.ds(base, _LPW)], idx_v.at[0])
    pltpu.sync_copy(xf_hbm.at[pl.ds(_L + base, _LPW)], idx_v.at[1])

    vmem = bufs[: 3 * _NSET]
    sems = bufs[3 * _NSET:]
    sets = tuple(
        (vmem[3 * s], vmem[3 * s + 1], vmem[3 * s + 2], sems[2 * s], sems[2 * s + 1])
        for s in range(_NSET)
    )

    def in_descs(c, s):
        rows0, rows1, posb, sem_in, _ = sets[s]
        g0 = pltpu.make_async_copy(
            table_hbm.at[idx_v.at[0, pl.ds(c * _CH, _CH)]], rows0, sem_in)
        g1 = pltpu.make_async_copy(
            table_hbm.at[idx_v.at[1, pl.ds(c * _CH, _CH)]], rows1, sem_in)
        p = pltpu.make_async_copy(
            pos_hbm.at[pl.ds(base + c * _CH, _CH)], posb, sem_in)
        return (g0, g1, p)

    def out_descs(c, s):
        rows0, rows1, _, _, sem_out = sets[s]
        o0 = pltpu.make_async_copy(
            rows0, out_hbm.at[pl.ds(base + c * _CH, _CH)], sem_out)
        o1 = pltpu.make_async_copy(
            rows1, out_hbm.at[pl.ds(_L + base + c * _CH, _CH)], sem_out)
        return (o0, o1)

    def add_pos(s):
        rows0, rows1, posb, _, _ = sets[s]

        def add_body(j, carry):
            col = j * _LANES
            for r in range(_CH):
                pv = posb[r, pl.ds(col, _LANES)]
                plsc.addupdate(rows0.at[r, pl.ds(col, _LANES)], pv)
                plsc.addupdate(rows1.at[r, pl.ds(col, _LANES)], pv)
            return carry

        lax.fori_loop(0, _D // _LANES, add_body, 0)

    for c0 in range(_LEAD):
        for d in in_descs(c0, c0):
            d.start()

    def quad_body(q, carry):
        for s in range(_NSET):
            c = q * _NSET + s
            s2 = (s + _LEAD) % _NSET

            @pl.when(c >= _LEAD)
            def _():
                for d in out_descs(c - _LEAD, s2):
                    d.wait()

            @pl.when(c + _LEAD < _NCHUNK)
            def _():
                for d in in_descs(c + _LEAD, s2):
                    d.start()

            for d in in_descs(c, s):
                d.wait()
            add_pos(s)
            for d in out_descs(c, s):
                d.start()
        return carry

    lax.fori_loop(0, _NCHUNK // _NSET, quad_body, 0)

    for c0 in range(_NCHUNK - _LEAD, _NCHUNK):
        for d in out_descs(c0, c0 % _NSET):
            d.wait()


def kernel(x, token_table):
    xf = x.reshape(-1).astype(jnp.int32)
    pos = jnp.asarray(_POS)
    out = _emb_lookup(xf, token_table, pos)
    return out.reshape(_B, _L, _D)
